# speedup vs baseline: 1.4142x; 1.4142x over previous
import functools

import jax
import jax.numpy as jnp
from jax import lax
from jax.experimental import pallas as pl
from jax.experimental.pallas import tpu as pltpu
from jax.experimental.pallas import tpu_sc as plsc

NUM_CORES = 2
NUM_SUBCORES = 16
NUM_WORKERS = NUM_CORES * NUM_SUBCORES
CHUNK = 32


def _emb_body(n_chunks, chunk, ids_hbm, table_hbm, out_hbm, idx_v, rows_v, sem):
    wid = lax.axis_index("s") * NUM_CORES + lax.axis_index("c")
    base = wid * (n_chunks * chunk)
    pltpu.sync_copy(ids_hbm.at[wid], idx_v)
    for j in range(n_chunks):
        pltpu.async_copy(table_hbm.at[idx_v.at[j]], rows_v, sem).wait()
        pltpu.sync_copy(rows_v, out_hbm.at[pl.ds(base + j * chunk, chunk)])


def kernel(input_ids, word_embeddings):
    b, s = input_ids.shape
    v, d = word_embeddings.shape
    n = b * s
    assert n % (NUM_WORKERS * CHUNK) == 0
    n_chunks = n // (NUM_WORKERS * CHUNK)
    ids = input_ids.reshape(NUM_WORKERS, n_chunks, CHUNK).astype(jnp.int32)

    mesh = plsc.VectorSubcoreMesh(core_axis_name="c", subcore_axis_name="s")
    emb = functools.partial(
        pl.kernel,
        out_type=jax.ShapeDtypeStruct((n, d), jnp.float32),
        mesh=mesh,
        scratch_types=[
            pltpu.VMEM((n_chunks, CHUNK), jnp.int32),
            pltpu.VMEM((CHUNK, d), jnp.float32),
            pltpu.SemaphoreType.DMA,
        ],
    )(functools.partial(_emb_body, n_chunks, CHUNK))
    out = emb(ids, word_embeddings)
    return out.reshape(b, s, d)


# speedup vs baseline: 1.6340x; 1.1554x over previous
import functools

import jax
import jax.numpy as jnp
from jax import lax
from jax.experimental import pallas as pl
from jax.experimental.pallas import tpu as pltpu
from jax.experimental.pallas import tpu_sc as plsc

NUM_CORES = 2
NUM_SUBCORES = 16
NUM_WORKERS = NUM_CORES * NUM_SUBCORES
CHUNK = 32


def _emb_body(n_chunks, chunk, ids_hbm, table_hbm, out_hbm, idx_v,
              rows0, rows1, g0, g1, o0, o1):
    wid = lax.axis_index("s") * NUM_CORES + lax.axis_index("c")
    base = wid * (n_chunks * chunk)
    pltpu.sync_copy(ids_hbm.at[wid], idx_v)
    rows, gsem = (rows0, rows1), (g0, g1)
    del o0, o1
    gd = [None] * n_chunks
    gd[0] = pltpu.async_copy(table_hbm.at[idx_v.at[0]], rows[0], gsem[0])
    for j in range(n_chunks):
        p = j % 2
        if j + 1 < n_chunks:
            q = (j + 1) % 2
            gd[j + 1] = pltpu.async_copy(
                table_hbm.at[idx_v.at[j + 1]], rows[q], gsem[q])
        gd[j].wait()
        pltpu.sync_copy(rows[p], out_hbm.at[pl.ds(base + j * chunk, chunk)])


def kernel(input_ids, word_embeddings):
    b, s = input_ids.shape
    v, d = word_embeddings.shape
    n = b * s
    assert n % (NUM_WORKERS * CHUNK) == 0
    n_chunks = n // (NUM_WORKERS * CHUNK)
    ids = input_ids.reshape(NUM_WORKERS, n_chunks, CHUNK).astype(jnp.int32)

    mesh = plsc.VectorSubcoreMesh(core_axis_name="c", subcore_axis_name="s")
    emb = functools.partial(
        pl.kernel,
        out_type=jax.ShapeDtypeStruct((n, d), jnp.float32),
        mesh=mesh,
        scratch_types=[
            pltpu.VMEM((n_chunks, CHUNK), jnp.int32),
            pltpu.VMEM((CHUNK, d), jnp.float32),
            pltpu.VMEM((CHUNK, d), jnp.float32),
            pltpu.SemaphoreType.DMA,
            pltpu.SemaphoreType.DMA,
            pltpu.SemaphoreType.DMA,
            pltpu.SemaphoreType.DMA,
        ],
    )(functools.partial(_emb_body, n_chunks, CHUNK))
    out = emb(ids, word_embeddings)
    return out.reshape(b, s, d)


# speedup vs baseline: 1.6596x; 1.0157x over previous
import functools

import jax
import jax.numpy as jnp
from jax import lax
from jax.experimental import pallas as pl
from jax.experimental.pallas import tpu as pltpu
from jax.experimental.pallas import tpu_sc as plsc

NUM_CORES = 2
NUM_SUBCORES = 16
NUM_WORKERS = NUM_CORES * NUM_SUBCORES
CHUNK = 56


def _emb_body(chunks, per_w, ids_hbm, table_hbm, out_hbm, idx_v,
              rows0, rows1, g0, g1, o0, o1):
    wid = lax.axis_index("s") * NUM_CORES + lax.axis_index("c")
    base = wid * per_w
    pltpu.sync_copy(ids_hbm.at[wid], idx_v)
    rows, gsem = (rows0, rows1), (g0, g1)
    del o0, o1
    n_chunks = len(chunks)
    offs = [sum(chunks[:j]) for j in range(n_chunks)]

    def gather(j, p):
        return pltpu.async_copy(
            table_hbm.at[idx_v.at[pl.ds(offs[j], chunks[j])]],
            rows[p].at[pl.ds(0, chunks[j])], gsem[p])

    gd = [None] * n_chunks
    gd[0] = gather(0, 0)
    for j in range(n_chunks):
        p = j % 2
        if j + 1 < n_chunks:
            gd[j + 1] = gather(j + 1, (j + 1) % 2)
        gd[j].wait()
        pltpu.sync_copy(rows[p].at[pl.ds(0, chunks[j])],
                        out_hbm.at[pl.ds(base + offs[j], chunks[j])])


def kernel(input_ids, word_embeddings):
    b, s = input_ids.shape
    v, d = word_embeddings.shape
    n = b * s
    assert n % NUM_WORKERS == 0
    per_w = n // NUM_WORKERS
    full, rem = divmod(per_w, CHUNK)
    chunks = [CHUNK] * full + ([rem] if rem else [])
    ids = input_ids.reshape(NUM_WORKERS, per_w).astype(jnp.int32)

    mesh = plsc.VectorSubcoreMesh(core_axis_name="c", subcore_axis_name="s")
    emb = functools.partial(
        pl.kernel,
        out_type=jax.ShapeDtypeStruct((n, d), jnp.float32),
        mesh=mesh,
        scratch_types=[
            pltpu.VMEM((per_w,), jnp.int32),
            pltpu.VMEM((CHUNK, d), jnp.float32),
            pltpu.VMEM((CHUNK, d), jnp.float32),
            pltpu.SemaphoreType.DMA,
            pltpu.SemaphoreType.DMA,
            pltpu.SemaphoreType.DMA,
            pltpu.SemaphoreType.DMA,
        ],
    )(functools.partial(_emb_body, chunks, per_w))
    out = emb(ids, word_embeddings)
    return out.reshape(b, s, d)
